# baseline (speedup 1.0000x reference)
import functools

import jax
import jax.numpy as jnp
from jax.experimental import pallas as pl

N_LOW = 2500
N_HIGH = 10000
C_IN = 256
C_SKIP = 128
C = 128
G = 8
DEPTH = 2
K = 16
EPS = 1e-5


_KNN_GRP = 256
_KNN_DEPTH = 4


def _knn_body(sq_ref, cp_ref, qt_ref, out_ref, *, npts, r):
    big = jnp.float32(jnp.inf)
    qt = qt_ref[...]
    cp = cp_ref[...]
    qsq = jnp.sum(qt * qt, axis=0, keepdims=True)
    prod = jax.lax.dot_general(cp, qt, (((1,), (0,)), ((), ())),
                               preferred_element_type=jnp.float32)
    d = sq_ref[...] + qsq - 2.0 * prod
    gsz = npts // _KNN_GRP
    d3 = d.reshape(_KNN_GRP, gsz, r)
    iota3 = (jax.lax.broadcasted_iota(jnp.int32, (_KNN_GRP, gsz, r), 0) * gsz
             + jax.lax.broadcasted_iota(jnp.int32, (_KNN_GRP, gsz, r), 1))
    vals = []
    idxs = []
    for _ in range(_KNN_DEPTH):
        g = jnp.min(d3, axis=1)
        e = d3 == g[:, None, :]
        gi = jnp.min(jnp.where(e, iota3, npts), axis=1)
        d3 = jnp.where(e, big, d3)
        vals.append(g)
        idxs.append(gi)
    cv = jnp.concatenate(vals, axis=0)
    ci = jnp.concatenate(idxs, axis=0)
    kio = jax.lax.broadcasted_iota(jnp.int32, (K, r), 0)
    acc = jnp.zeros((K, r), jnp.int32)
    for t in range(K):
        m = jnp.min(cv, axis=0, keepdims=True)
        cand = jnp.where(cv == m, ci, npts)
        j = jnp.min(cand, axis=0, keepdims=True)
        acc = jnp.where(kio == t, jnp.broadcast_to(j, (K, r)), acc)
        cv = jnp.where(cand == j, big, cv)
    out_ref[...] = acc


def _knn(coord):
    n = coord.shape[0]
    r = 128
    npad = 10240 if n == N_HIGH else ((n + 1279) // 1280) * 1280
    nqp = ((n + r - 1) // r) * r
    pad = jnp.full((npad - n, 3), 1e4, jnp.float32)
    cp0 = jnp.concatenate([coord, pad], axis=0)
    cp = jnp.concatenate([cp0, jnp.zeros((npad, 5), jnp.float32)], axis=1)
    qt = jnp.concatenate([cp0[:nqp].T, jnp.zeros((5, nqp), jnp.float32)], axis=0)
    sq = jnp.sum(cp0 * cp0, axis=1)[:, None]
    grid = nqp // r
    body = functools.partial(_knn_body, npts=npad, r=r)
    idx_t = pl.pallas_call(
        body,
        grid=(grid,),
        in_specs=[
            pl.BlockSpec((npad, 1), lambda i: (0, 0)),
            pl.BlockSpec((npad, 8), lambda i: (0, 0)),
            pl.BlockSpec((8, r), lambda i: (0, i)),
        ],
        out_specs=pl.BlockSpec((K, r), lambda i: (0, i)),
        out_shape=jax.ShapeDtypeStruct((K, nqp), jnp.int32),
    )(sq, cp, qt)
    return idx_t.T[:n]


NLP = 2560
NHP = 10240
TN = 128
NPAIR = N_HIGH * K
GRID_P = NHP // TN


def _bnm(y, gam, bet, nvalid):
    nr = y.shape[0]
    valid = jax.lax.broadcasted_iota(jnp.int32, (nr, 1), 0) < nvalid
    ym = jnp.where(valid, y, 0.0)
    mu = jnp.sum(ym, axis=0, keepdims=True) / nvalid
    dev = jnp.where(valid, y - mu, 0.0)
    var = jnp.sum(dev * dev, axis=0, keepdims=True) / nvalid
    return (y - mu) / jnp.sqrt(var + EPS) * gam + bet


def _pre_body(featp, skipp, wupt, bup, gup, betup, wskt, bsk, gsk, betsk,
              f0_ref, sf_ref):
    y0 = jax.lax.dot(featp[...], wupt[...],
                     preferred_element_type=jnp.float32) + bup[...]
    f0_ref[...] = jax.nn.relu(_bnm(y0, gup[...], betup[...], N_LOW))
    y1 = jax.lax.dot(skipp[...], wskt[...],
                     preferred_element_type=jnp.float32) + bsk[...]
    sf_ref[...] = jax.nn.relu(_bnm(y1, gsk[...], betsk[...], N_HIGH))


def _d1_body(fcl, sf, xyz8, wfc1t, g1, b1, wqt, bq, gq, bqn, wkt, bk, gk, bkn,
             wvt, bv, we1t, f_ref, v_ref, q8_ref, t_ref):
    f = fcl[...] + sf[...]
    f_ref[...] = f
    f1 = jax.nn.relu(_bnm(jax.lax.dot(f, wfc1t[...],
                                      preferred_element_type=jnp.float32),
                          g1[...], b1[...], N_HIGH))
    q = jax.nn.relu(_bnm(jax.lax.dot(f1, wqt[...],
                                     preferred_element_type=jnp.float32) + bq[...],
                         gq[...], bqn[...], N_HIGH))
    k = jax.nn.relu(_bnm(jax.lax.dot(f1, wkt[...],
                                     preferred_element_type=jnp.float32) + bk[...],
                         gk[...], bkn[...], N_HIGH))
    v_ref[...] = jax.lax.dot(f1, wvt[...],
                             preferred_element_type=jnp.float32) + bv[...]
    q8_ref[...] = jax.lax.dot(q, we1t[...], preferred_element_type=jnp.float32)
    k8 = jax.lax.dot(k, we1t[...], preferred_element_type=jnp.float32)
    t_ref[...] = jnp.concatenate(
        [xyz8[...][:, 0:3], k8, jnp.zeros((NHP, 5), jnp.float32)], axis=1)


def _pstat_body(g3, xyz8, o_ref):
    i = pl.program_id(0)
    pos = g3[...][:, :, 0:3] - xyz8[...][:, None, 0:3]
    valid = (jax.lax.broadcasted_iota(jnp.int32, (TN, 1, 1), 0)
             + i * TN) < N_HIGH
    pos = jnp.where(valid, pos, 0.0)
    pos2 = pos.reshape(TN * K, 3)
    s1 = jnp.sum(pos2, axis=0, keepdims=True)
    s2 = jax.lax.dot_general(pos2, pos2, (((0,), (0,)), ((), ())),
                             preferred_element_type=jnp.float32)
    part = jnp.concatenate([
        jnp.pad(s1, ((0, 0), (0, 125))),
        jnp.pad(s2, ((0, 0), (0, 125))),
        jnp.zeros((4, 128), jnp.float32)], axis=0)

    @pl.when(i == 0)
    def _():
        o_ref[...] = part

    @pl.when(i > 0)
    def _():
        o_ref[...] = o_ref[...] + part


def _pairs_front(g3, xyz8, q8, a8, ca, m128, c8):
    pos = g3[:, :, 0:3] - xyz8[:, None, 0:3]
    h = jax.nn.relu(jax.lax.dot(pos.reshape(TN * K, 3), a8[0:3, :],
                                preferred_element_type=jnp.float32) + ca)
    k8g = g3[:, :, 3:11].reshape(TN * K, 8)
    q8r = jnp.broadcast_to(q8[:, None, :], (TN, K, 8)).reshape(TN * K, 8)
    z8 = k8g - q8r + jax.lax.dot(h, m128[...],
                                 preferred_element_type=jnp.float32) + c8
    return h, z8


def _zstat_body(g3, xyz8, q8, a8, ca, m128, c8, o_ref):
    i = pl.program_id(0)
    _, z8 = _pairs_front(g3[...], xyz8[...], q8[...], a8[...], ca[...],
                         m128, c8[...])
    valid = (jax.lax.broadcasted_iota(jnp.int32, (TN * K, 1), 0)
             // K + i * TN) < N_HIGH
    z8 = jnp.where(valid, z8, 0.0)
    s1 = jnp.sum(z8, axis=0, keepdims=True)
    s2 = jnp.sum(z8 * z8, axis=0, keepdims=True)
    part = jnp.concatenate([
        jnp.pad(s1, ((0, 0), (0, 120))),
        jnp.pad(s2, ((0, 0), (0, 120))),
        jnp.zeros((6, 128), jnp.float32)], axis=0)

    @pl.when(i == 0)
    def _():
        o_ref[...] = part

    @pl.when(i > 0)
    def _():
        o_ref[...] = o_ref[...] + part


def _agg_body(g3, gv, xyz8, q8, a8, ca, m128, c8, sw, tw, we2t, bwe2,
              p2t, bp2, e8, f2_ref):
    h, z8 = _pairs_front(g3[...], xyz8[...], q8[...], a8[...], ca[...],
                         m128, c8[...])
    lg = jax.lax.dot(jax.nn.relu(z8 * sw[...] + tw[...]), we2t[...],
                     preferred_element_type=jnp.float32) + bwe2[...]
    l3 = lg.reshape(TN, K, 8)
    mx = jnp.max(l3, axis=1, keepdims=True)
    ex = jnp.exp(l3 - mx)
    w3 = ex / jnp.sum(ex, axis=1, keepdims=True)
    wexp = jax.lax.dot(w3.reshape(TN * K, 8), e8[...],
                       preferred_element_type=jnp.float32)
    gv2 = gv[...].reshape(TN * K, C)
    seg = jnp.sum((wexp * gv2).reshape(TN, K, C), axis=1)
    h3 = h.reshape(TN, K, C)
    pieces = []
    for g in range(G):
        hg = jnp.sum(w3[:, :, g:g + 1] * h3, axis=1)
        pieces.append(jax.lax.dot(hg, p2t[...][:, g * 16:(g + 1) * 16],
                                  preferred_element_type=jnp.float32))
    f2_ref[...] = seg + jnp.concatenate(pieces, axis=1) + bp2[...]


def _d3_body(f, f2, wfc3t, g2, b2, g3n, b3n, out_ref):
    y2 = jax.nn.relu(_bnm(f2[...], g2[...], b2[...], N_HIGH))
    y3 = _bnm(jax.lax.dot(y2, wfc3t[...], preferred_element_type=jnp.float32),
              g3n[...], b3n[...], N_HIGH)
    out_ref[...] = jax.nn.relu(f[...] + y3)


def _padr(x, rows):
    return jnp.concatenate(
        [x, jnp.zeros((rows - x.shape[0],) + x.shape[1:], x.dtype)], axis=0)


def _full_specs(shapes):
    def mk(n):
        return lambda i: (0,) * n
    return [pl.BlockSpec(s, mk(len(s))) for s in shapes]


def _gather_rows(table, idxfp):
    return table[idxfp]


def kernel(coord, feat, offset, skip_coord, skip_feat, skip_offset, cluster, params):
    ref = _knn(skip_coord)

    xyz8 = jnp.concatenate(
        [jnp.concatenate([skip_coord, jnp.full((NHP - N_HIGH, 3), 1e4,
                                               jnp.float32)], axis=0),
         jnp.zeros((NHP, 5), jnp.float32)], axis=1)
    idxfp = jnp.concatenate(
        [ref.reshape(-1), jnp.zeros((NHP * K - NPAIR,), jnp.int32)])

    featp = _padr(feat, NLP)
    skipp = _padr(skip_feat, NHP)
    p = params
    r1 = lambda a: a.reshape(1, -1)
    f0, sf = pl.pallas_call(
        _pre_body,
        out_shape=[jax.ShapeDtypeStruct((NLP, C), jnp.float32),
                   jax.ShapeDtypeStruct((NHP, C), jnp.float32)],
    )(featp, skipp,
      p["up_proj"]["W"].T, r1(p["up_proj"]["b"]),
      r1(p["up_proj_bn"]["g"]), r1(p["up_proj_bn"]["b"]),
      p["up_skip"]["W"].T, r1(p["up_skip"]["b"]),
      r1(p["up_skip_bn"]["g"]), r1(p["up_skip_bn"]["b"]))

    clusterp = _padr(cluster, NHP)
    fcl = _gather_rows(f0, clusterp)

    f = None
    pstats = None
    for bi, blk in enumerate(p["blocks"]):
        we1t = blk["we1"]["W"].T
        if bi == 0:
            fin_a, fin_b = fcl, sf
        else:
            fin_a, fin_b = f, jnp.zeros_like(f)
        f, v, q8, tab = pl.pallas_call(
            _d1_body,
            out_shape=[jax.ShapeDtypeStruct((NHP, C), jnp.float32),
                       jax.ShapeDtypeStruct((NHP, C), jnp.float32),
                       jax.ShapeDtypeStruct((NHP, 8), jnp.float32),
                       jax.ShapeDtypeStruct((NHP, 16), jnp.float32)],
        )(fin_a, fin_b, xyz8,
          blk["fc1"]["W"].T, r1(blk["norm1"]["g"]), r1(blk["norm1"]["b"]),
          blk["q"]["W"].T, r1(blk["q"]["b"]),
          r1(blk["q_bn"]["g"]), r1(blk["q_bn"]["b"]),
          blk["k"]["W"].T, r1(blk["k"]["b"]),
          r1(blk["k_bn"]["g"]), r1(blk["k_bn"]["b"]),
          blk["v"]["W"].T, r1(blk["v"]["b"]), we1t)

        g16 = _gather_rows(tab, idxfp).reshape(NHP, K, 16)
        gv = _gather_rows(v, idxfp).reshape(NHP, K, C)

        if pstats is None:
            pstats = pl.pallas_call(
                _pstat_body,
                grid=(GRID_P,),
                in_specs=[pl.BlockSpec((TN, K, 16), lambda i: (i, 0, 0)),
                          pl.BlockSpec((TN, 8), lambda i: (i, 0))],
                out_specs=pl.BlockSpec((8, 128), lambda i: (0, 0)),
                out_shape=jax.ShapeDtypeStruct((8, 128), jnp.float32),
            )(g16, xyz8)
            s1 = pstats[0, 0:3]
            s2 = pstats[1:4, 0:3]
            mpos = s1 / NPAIR
            cov = s2 / NPAIR - jnp.outer(mpos, mpos)

        p1t = blk["p1"]["W"].T
        mu_p = mpos @ p1t + blk["p1"]["b"]
        var_p = jnp.einsum("ij,ic,jc->c", cov, p1t, p1t)
        sg = blk["p_bn"]["g"] / jnp.sqrt(var_p + EPS)
        a3 = p1t * sg[None, :]
        a8 = jnp.concatenate([a3, jnp.zeros((5, C), jnp.float32)], axis=0)
        ca = ((blk["p1"]["b"] - mu_p) * sg + blk["p_bn"]["b"]).reshape(1, C)
        m128 = blk["p2"]["W"].T @ we1t
        c8 = (blk["p2"]["b"] @ we1t + blk["we1"]["b"]).reshape(1, 8)

        zstats = pl.pallas_call(
            _zstat_body,
            grid=(GRID_P,),
            in_specs=[pl.BlockSpec((TN, K, 16), lambda i: (i, 0, 0)),
                      pl.BlockSpec((TN, 8), lambda i: (i, 0)),
                      pl.BlockSpec((TN, 8), lambda i: (i, 0))]
                     + _full_specs([(8, C), (1, C), (C, 8), (1, 8)]),
            out_specs=pl.BlockSpec((8, 128), lambda i: (0, 0)),
            out_shape=jax.ShapeDtypeStruct((8, 128), jnp.float32),
        )(g16, xyz8, q8, a8, ca, m128, c8)
        mu_z = zstats[0, 0:8] / NPAIR
        var_z = zstats[1, 0:8] / NPAIR - mu_z * mu_z
        sw = (blk["we_bn"]["g"] / jnp.sqrt(var_z + EPS)).reshape(1, 8)
        tw = (blk["we_bn"]["b"] - mu_z * sw[0]).reshape(1, 8)

        e8 = (jnp.arange(C)[None, :] // 16 ==
              jnp.arange(G)[:, None]).astype(jnp.float32)
        f2 = pl.pallas_call(
            _agg_body,
            grid=(GRID_P,),
            in_specs=[pl.BlockSpec((TN, K, 16), lambda i: (i, 0, 0)),
                      pl.BlockSpec((TN, K, C), lambda i: (i, 0, 0)),
                      pl.BlockSpec((TN, 8), lambda i: (i, 0)),
                      pl.BlockSpec((TN, 8), lambda i: (i, 0))]
                     + _full_specs([(8, C), (1, C), (C, 8), (1, 8), (1, 8),
                                    (1, 8), (8, 8), (1, 8), (C, C), (1, C),
                                    (8, C)]),
            out_specs=pl.BlockSpec((TN, C), lambda i: (i, 0)),
            out_shape=jax.ShapeDtypeStruct((NHP, C), jnp.float32),
        )(g16, gv, xyz8, q8, a8, ca, m128, c8, sw, tw,
          blk["we2"]["W"].T, r1(blk["we2"]["b"]), blk["p2"]["W"].T,
          r1(blk["p2"]["b"]), e8)

        f = pl.pallas_call(
            _d3_body,
            out_shape=jax.ShapeDtypeStruct((NHP, C), jnp.float32),
        )(f, f2, blk["fc3"]["W"].T,
          r1(blk["norm2"]["g"]), r1(blk["norm2"]["b"]),
          r1(blk["norm3"]["g"]), r1(blk["norm3"]["b"]))

    return (skip_coord, f[:N_HIGH], skip_offset, ref)


# speedup vs baseline: 2.0477x; 2.0477x over previous
import functools

import jax
import jax.numpy as jnp
from jax.experimental import pallas as pl
from jax.experimental.pallas import tpu as pltpu
from jax.experimental.pallas import tpu_sc as plsc

N_LOW = 2500
N_HIGH = 10000
C_IN = 256
C_SKIP = 128
C = 128
G = 8
DEPTH = 2
K = 16
EPS = 1e-5


_KNN_GRP = 256
_KNN_DEPTH = 4


def _knn_body(sq_ref, cp_ref, qt_ref, out_ref, *, npts, r):
    big = jnp.float32(jnp.inf)
    qt = qt_ref[...]
    cp = cp_ref[...]
    qsq = jnp.sum(qt * qt, axis=0, keepdims=True)
    prod = jax.lax.dot_general(cp, qt, (((1,), (0,)), ((), ())),
                               preferred_element_type=jnp.float32)
    d = sq_ref[...] + qsq - 2.0 * prod
    gsz = npts // _KNN_GRP
    d3 = d.reshape(_KNN_GRP, gsz, r)
    iota3 = (jax.lax.broadcasted_iota(jnp.int32, (_KNN_GRP, gsz, r), 0) * gsz
             + jax.lax.broadcasted_iota(jnp.int32, (_KNN_GRP, gsz, r), 1))
    vals = []
    idxs = []
    for _ in range(_KNN_DEPTH):
        g = jnp.min(d3, axis=1)
        e = d3 == g[:, None, :]
        gi = jnp.min(jnp.where(e, iota3, npts), axis=1)
        d3 = jnp.where(e, big, d3)
        vals.append(g)
        idxs.append(gi)
    cv = jnp.concatenate(vals, axis=0)
    ci = jnp.concatenate(idxs, axis=0)
    kio = jax.lax.broadcasted_iota(jnp.int32, (K, r), 0)
    acc = jnp.zeros((K, r), jnp.int32)
    for t in range(K):
        m = jnp.min(cv, axis=0, keepdims=True)
        cand = jnp.where(cv == m, ci, npts)
        j = jnp.min(cand, axis=0, keepdims=True)
        acc = jnp.where(kio == t, jnp.broadcast_to(j, (K, r)), acc)
        cv = jnp.where(cand == j, big, cv)
    out_ref[...] = acc


def _knn(coord):
    n = coord.shape[0]
    r = 128
    npad = 10240 if n == N_HIGH else ((n + 1279) // 1280) * 1280
    nqp = ((n + r - 1) // r) * r
    pad = jnp.full((npad - n, 3), 1e4, jnp.float32)
    cp0 = jnp.concatenate([coord, pad], axis=0)
    cp = jnp.concatenate([cp0, jnp.zeros((npad, 5), jnp.float32)], axis=1)
    qt = jnp.concatenate([cp0[:nqp].T, jnp.zeros((5, nqp), jnp.float32)], axis=0)
    sq = jnp.sum(cp0 * cp0, axis=1)[:, None]
    grid = nqp // r
    body = functools.partial(_knn_body, npts=npad, r=r)
    idx_t = pl.pallas_call(
        body,
        grid=(grid,),
        in_specs=[
            pl.BlockSpec((npad, 1), lambda i: (0, 0)),
            pl.BlockSpec((npad, 8), lambda i: (0, 0)),
            pl.BlockSpec((8, r), lambda i: (0, i)),
        ],
        out_specs=pl.BlockSpec((K, r), lambda i: (0, i)),
        out_shape=jax.ShapeDtypeStruct((K, nqp), jnp.int32),
    )(sq, cp, qt)
    return idx_t.T[:n]


NLP = 2560
NHP = 10240
TN = 128
NPAIR = N_HIGH * K
GRID_P = NHP // TN


def _bnm(y, gam, bet, nvalid):
    nr = y.shape[0]
    valid = jax.lax.broadcasted_iota(jnp.int32, (nr, 1), 0) < nvalid
    ym = jnp.where(valid, y, 0.0)
    mu = jnp.sum(ym, axis=0, keepdims=True) / nvalid
    dev = jnp.where(valid, y - mu, 0.0)
    var = jnp.sum(dev * dev, axis=0, keepdims=True) / nvalid
    return (y - mu) / jnp.sqrt(var + EPS) * gam + bet


def _pre_body(featp, skipp, wupt, bup, gup, betup, wskt, bsk, gsk, betsk,
              f0_ref, sf_ref):
    y0 = jax.lax.dot(featp[...], wupt[...],
                     preferred_element_type=jnp.float32) + bup[...]
    f0_ref[...] = jax.nn.relu(_bnm(y0, gup[...], betup[...], N_LOW))
    y1 = jax.lax.dot(skipp[...], wskt[...],
                     preferred_element_type=jnp.float32) + bsk[...]
    sf_ref[...] = jax.nn.relu(_bnm(y1, gsk[...], betsk[...], N_HIGH))


def _d1_body(fcl, sf, xyz8, wfc1t, g1, b1, wqt, bq, gq, bqn, wkt, bk, gk, bkn,
             wvt, bv, we1t, f_ref, v_ref, q8_ref, t_ref):
    f = fcl[...] + sf[...]
    f_ref[...] = f
    f1 = jax.nn.relu(_bnm(jax.lax.dot(f, wfc1t[...],
                                      preferred_element_type=jnp.float32),
                          g1[...], b1[...], N_HIGH))
    q = jax.nn.relu(_bnm(jax.lax.dot(f1, wqt[...],
                                     preferred_element_type=jnp.float32) + bq[...],
                         gq[...], bqn[...], N_HIGH))
    k = jax.nn.relu(_bnm(jax.lax.dot(f1, wkt[...],
                                     preferred_element_type=jnp.float32) + bk[...],
                         gk[...], bkn[...], N_HIGH))
    v_ref[...] = jax.lax.dot(f1, wvt[...],
                             preferred_element_type=jnp.float32) + bv[...]
    q8_ref[...] = jax.lax.dot(q, we1t[...], preferred_element_type=jnp.float32)
    k8 = jax.lax.dot(k, we1t[...], preferred_element_type=jnp.float32)
    t_ref[...] = jnp.concatenate(
        [xyz8[...][:, 0:3], k8, jnp.zeros((NHP, C - 11), jnp.float32)], axis=1)


def _pstat_body(g3, xyz8, o_ref):
    i = pl.program_id(0)
    pos = g3[...][:, :, 0:3] - xyz8[...][:, None, 0:3]
    valid = (jax.lax.broadcasted_iota(jnp.int32, (TN, 1, 1), 0)
             + i * TN) < N_HIGH
    pos = jnp.where(valid, pos, 0.0)
    pos2 = pos.reshape(TN * K, 3)
    s1 = jnp.sum(pos2, axis=0, keepdims=True)
    s2 = jax.lax.dot_general(pos2, pos2, (((0,), (0,)), ((), ())),
                             preferred_element_type=jnp.float32)
    part = jnp.concatenate([
        jnp.pad(s1, ((0, 0), (0, 125))),
        jnp.pad(s2, ((0, 0), (0, 125))),
        jnp.zeros((4, 128), jnp.float32)], axis=0)

    @pl.when(i == 0)
    def _():
        o_ref[...] = part

    @pl.when(i > 0)
    def _():
        o_ref[...] = o_ref[...] + part


def _pairs_front(g3, xyz8, q8, a8, ca, m128, c8):
    pos = g3[:, :, 0:3] - xyz8[:, None, 0:3]
    h = jax.nn.relu(jax.lax.dot(pos.reshape(TN * K, 3), a8[0:3, :],
                                preferred_element_type=jnp.float32) + ca)
    k8g = g3[:, :, 3:11].reshape(TN * K, 8)
    q8r = jnp.broadcast_to(q8[:, None, :], (TN, K, 8)).reshape(TN * K, 8)
    z8 = k8g - q8r + jax.lax.dot(h, m128[...],
                                 preferred_element_type=jnp.float32) + c8
    return h, z8


def _zstat_body(g3, xyz8, q8, a8, ca, m128, c8, o_ref):
    i = pl.program_id(0)
    _, z8 = _pairs_front(g3[...], xyz8[...], q8[...], a8[...], ca[...],
                         m128, c8[...])
    valid = (jax.lax.broadcasted_iota(jnp.int32, (TN * K, 1), 0)
             // K + i * TN) < N_HIGH
    z8 = jnp.where(valid, z8, 0.0)
    s1 = jnp.sum(z8, axis=0, keepdims=True)
    s2 = jnp.sum(z8 * z8, axis=0, keepdims=True)
    part = jnp.concatenate([
        jnp.pad(s1, ((0, 0), (0, 120))),
        jnp.pad(s2, ((0, 0), (0, 120))),
        jnp.zeros((6, 128), jnp.float32)], axis=0)

    @pl.when(i == 0)
    def _():
        o_ref[...] = part

    @pl.when(i > 0)
    def _():
        o_ref[...] = o_ref[...] + part


def _agg_body(g3, gv, xyz8, q8, a8, ca, m128, c8, sw, tw, we2t, bwe2,
              p2t, bp2, e8, f2_ref):
    h, z8 = _pairs_front(g3[...], xyz8[...], q8[...], a8[...], ca[...],
                         m128, c8[...])
    lg = jax.lax.dot(jax.nn.relu(z8 * sw[...] + tw[...]), we2t[...],
                     preferred_element_type=jnp.float32) + bwe2[...]
    l3 = lg.reshape(TN, K, 8)
    mx = jnp.max(l3, axis=1, keepdims=True)
    ex = jnp.exp(l3 - mx)
    w3 = ex / jnp.sum(ex, axis=1, keepdims=True)
    wexp = jax.lax.dot(w3.reshape(TN * K, 8), e8[...],
                       preferred_element_type=jnp.float32)
    gv2 = gv[...].reshape(TN * K, C)
    seg = jnp.sum((wexp * gv2).reshape(TN, K, C), axis=1)
    h3 = h.reshape(TN, K, C)
    pieces = []
    for g in range(G):
        hg = jnp.sum(w3[:, :, g:g + 1] * h3, axis=1)
        pieces.append(jax.lax.dot(hg, p2t[...][:, g * 16:(g + 1) * 16],
                                  preferred_element_type=jnp.float32))
    f2_ref[...] = seg + jnp.concatenate(pieces, axis=1) + bp2[...]


def _d3_body(f, f2, wfc3t, g2, b2, g3n, b3n, out_ref):
    y2 = jax.nn.relu(_bnm(f2[...], g2[...], b2[...], N_HIGH))
    y3 = _bnm(jax.lax.dot(y2, wfc3t[...], preferred_element_type=jnp.float32),
              g3n[...], b3n[...], N_HIGH)
    out_ref[...] = jax.nn.relu(f[...] + y3)


def _padr(x, rows):
    return jnp.concatenate(
        [x, jnp.zeros((rows - x.shape[0],) + x.shape[1:], x.dtype)], axis=0)


def _full_specs(shapes):
    def mk(n):
        return lambda i: (0,) * n
    return [pl.BlockSpec(s, mk(len(s))) for s in shapes]


_SC_NW = 32


def _gather_rows(table, idx):
    b = idx.shape[0]
    d = table.shape[1]
    bpw = b // _SC_NW
    bc = bpw
    while bc * d * 4 > 400_000:
        bc //= 2
    nch = bpw // bc
    mesh = plsc.VectorSubcoreMesh(core_axis_name="c", subcore_axis_name="s")

    @functools.partial(
        pl.kernel, mesh=mesh,
        out_type=jax.ShapeDtypeStruct((b, d), jnp.float32),
        scratch_types=[
            pltpu.VMEM((bc,), jnp.int32),
            pltpu.VMEM((bc, d), jnp.float32),
            pltpu.SemaphoreType.DMA,
        ],
    )
    def gk(table_hbm, idx_hbm, out_hbm, idx_v, rows_v, sem):
        wid = jax.lax.axis_index("s") * 2 + jax.lax.axis_index("c")
        for j in range(nch):
            base = wid * bpw + j * bc
            pltpu.sync_copy(idx_hbm.at[pl.ds(base, bc)], idx_v)
            pltpu.async_copy(table_hbm.at[idx_v], rows_v, sem).wait()
            pltpu.sync_copy(rows_v, out_hbm.at[pl.ds(base, bc)])

    return gk(table, idx)


def kernel(coord, feat, offset, skip_coord, skip_feat, skip_offset, cluster, params):
    ref = _knn(skip_coord)

    xyz8 = jnp.concatenate(
        [jnp.concatenate([skip_coord, jnp.full((NHP - N_HIGH, 3), 1e4,
                                               jnp.float32)], axis=0),
         jnp.zeros((NHP, 5), jnp.float32)], axis=1)
    idxfp = jnp.concatenate(
        [ref.reshape(-1),
         jnp.arange(NHP * K - NPAIR, dtype=jnp.int32) % N_HIGH])

    featp = _padr(feat, NLP)
    skipp = _padr(skip_feat, NHP)
    p = params
    r1 = lambda a: a.reshape(1, -1)
    f0, sf = pl.pallas_call(
        _pre_body,
        out_shape=[jax.ShapeDtypeStruct((NLP, C), jnp.float32),
                   jax.ShapeDtypeStruct((NHP, C), jnp.float32)],
    )(featp, skipp,
      p["up_proj"]["W"].T, r1(p["up_proj"]["b"]),
      r1(p["up_proj_bn"]["g"]), r1(p["up_proj_bn"]["b"]),
      p["up_skip"]["W"].T, r1(p["up_skip"]["b"]),
      r1(p["up_skip_bn"]["g"]), r1(p["up_skip_bn"]["b"]))

    clusterp = jnp.concatenate(
        [cluster, jnp.arange(NHP - N_HIGH, dtype=jnp.int32) % N_LOW])
    fcl = _gather_rows(f0, clusterp)

    f = None
    pstats = None
    for bi, blk in enumerate(p["blocks"]):
        we1t = blk["we1"]["W"].T
        if bi == 0:
            fin_a, fin_b = fcl, sf
        else:
            fin_a, fin_b = f, jnp.zeros_like(f)
        f, v, q8, tab = pl.pallas_call(
            _d1_body,
            out_shape=[jax.ShapeDtypeStruct((NHP, C), jnp.float32),
                       jax.ShapeDtypeStruct((NHP, C), jnp.float32),
                       jax.ShapeDtypeStruct((NHP, 8), jnp.float32),
                       jax.ShapeDtypeStruct((NHP, C), jnp.float32)],
        )(fin_a, fin_b, xyz8,
          blk["fc1"]["W"].T, r1(blk["norm1"]["g"]), r1(blk["norm1"]["b"]),
          blk["q"]["W"].T, r1(blk["q"]["b"]),
          r1(blk["q_bn"]["g"]), r1(blk["q_bn"]["b"]),
          blk["k"]["W"].T, r1(blk["k"]["b"]),
          r1(blk["k_bn"]["g"]), r1(blk["k_bn"]["b"]),
          blk["v"]["W"].T, r1(blk["v"]["b"]), we1t)

        g16 = _gather_rows(tab, idxfp).reshape(NHP, K, C)
        gv = _gather_rows(v, idxfp).reshape(NHP, K, C)

        if pstats is None:
            pstats = pl.pallas_call(
                _pstat_body,
                grid=(GRID_P,),
                in_specs=[pl.BlockSpec((TN, K, C), lambda i: (i, 0, 0)),
                          pl.BlockSpec((TN, 8), lambda i: (i, 0))],
                out_specs=pl.BlockSpec((8, 128), lambda i: (0, 0)),
                out_shape=jax.ShapeDtypeStruct((8, 128), jnp.float32),
            )(g16, xyz8)
            s1 = pstats[0, 0:3]
            s2 = pstats[1:4, 0:3]
            mpos = s1 / NPAIR
            cov = s2 / NPAIR - jnp.outer(mpos, mpos)

        p1t = blk["p1"]["W"].T
        mu_p = mpos @ p1t + blk["p1"]["b"]
        var_p = jnp.einsum("ij,ic,jc->c", cov, p1t, p1t)
        sg = blk["p_bn"]["g"] / jnp.sqrt(var_p + EPS)
        a3 = p1t * sg[None, :]
        a8 = jnp.concatenate([a3, jnp.zeros((5, C), jnp.float32)], axis=0)
        ca = ((blk["p1"]["b"] - mu_p) * sg + blk["p_bn"]["b"]).reshape(1, C)
        m128 = blk["p2"]["W"].T @ we1t
        c8 = (blk["p2"]["b"] @ we1t + blk["we1"]["b"]).reshape(1, 8)

        zstats = pl.pallas_call(
            _zstat_body,
            grid=(GRID_P,),
            in_specs=[pl.BlockSpec((TN, K, C), lambda i: (i, 0, 0)),
                      pl.BlockSpec((TN, 8), lambda i: (i, 0)),
                      pl.BlockSpec((TN, 8), lambda i: (i, 0))]
                     + _full_specs([(8, C), (1, C), (C, 8), (1, 8)]),
            out_specs=pl.BlockSpec((8, 128), lambda i: (0, 0)),
            out_shape=jax.ShapeDtypeStruct((8, 128), jnp.float32),
        )(g16, xyz8, q8, a8, ca, m128, c8)
        mu_z = zstats[0, 0:8] / NPAIR
        var_z = zstats[1, 0:8] / NPAIR - mu_z * mu_z
        sw = (blk["we_bn"]["g"] / jnp.sqrt(var_z + EPS)).reshape(1, 8)
        tw = (blk["we_bn"]["b"] - mu_z * sw[0]).reshape(1, 8)

        e8 = (jnp.arange(C)[None, :] // 16 ==
              jnp.arange(G)[:, None]).astype(jnp.float32)
        f2 = pl.pallas_call(
            _agg_body,
            grid=(GRID_P,),
            in_specs=[pl.BlockSpec((TN, K, C), lambda i: (i, 0, 0)),
                      pl.BlockSpec((TN, K, C), lambda i: (i, 0, 0)),
                      pl.BlockSpec((TN, 8), lambda i: (i, 0)),
                      pl.BlockSpec((TN, 8), lambda i: (i, 0))]
                     + _full_specs([(8, C), (1, C), (C, 8), (1, 8), (1, 8),
                                    (1, 8), (8, 8), (1, 8), (C, C), (1, C),
                                    (8, C)]),
            out_specs=pl.BlockSpec((TN, C), lambda i: (i, 0)),
            out_shape=jax.ShapeDtypeStruct((NHP, C), jnp.float32),
        )(g16, gv, xyz8, q8, a8, ca, m128, c8, sw, tw,
          blk["we2"]["W"].T, r1(blk["we2"]["b"]), blk["p2"]["W"].T,
          r1(blk["p2"]["b"]), e8)

        f = pl.pallas_call(
            _d3_body,
            out_shape=jax.ShapeDtypeStruct((NHP, C), jnp.float32),
        )(f, f2, blk["fc3"]["W"].T,
          r1(blk["norm2"]["g"]), r1(blk["norm2"]["b"]),
          r1(blk["norm3"]["g"]), r1(blk["norm3"]["b"]))

    return (skip_coord, f[:N_HIGH], skip_offset, ref)


# speedup vs baseline: 2.4743x; 1.2084x over previous
import functools

import jax
import jax.numpy as jnp
from jax.experimental import pallas as pl
from jax.experimental.pallas import tpu as pltpu
from jax.experimental.pallas import tpu_sc as plsc

N_LOW = 2500
N_HIGH = 10000
C_IN = 256
C_SKIP = 128
C = 128
G = 8
DEPTH = 2
K = 16
EPS = 1e-5


_KNN_GRP = 256
_KNN_DEPTH = 4


def _knn_body(sq_ref, cp_ref, qt_ref, out_ref, *, npts, r):
    big = jnp.float32(jnp.inf)
    qt = qt_ref[...]
    cp = cp_ref[...]
    qsq = jnp.sum(qt * qt, axis=0, keepdims=True)
    prod = jax.lax.dot_general(cp, qt, (((1,), (0,)), ((), ())),
                               preferred_element_type=jnp.float32)
    d = sq_ref[...] + qsq - 2.0 * prod
    gsz = npts // _KNN_GRP
    d3 = d.reshape(_KNN_GRP, gsz, r)
    iota3 = (jax.lax.broadcasted_iota(jnp.int32, (_KNN_GRP, gsz, r), 0) * gsz
             + jax.lax.broadcasted_iota(jnp.int32, (_KNN_GRP, gsz, r), 1))
    vals = []
    idxs = []
    for _ in range(_KNN_DEPTH):
        g = jnp.min(d3, axis=1)
        e = d3 == g[:, None, :]
        gi = jnp.min(jnp.where(e, iota3, npts), axis=1)
        d3 = jnp.where(e, big, d3)
        vals.append(g)
        idxs.append(gi)
    cv = jnp.concatenate(vals, axis=0)
    ci = jnp.concatenate(idxs, axis=0)
    kio = jax.lax.broadcasted_iota(jnp.int32, (K, r), 0)
    acc = jnp.zeros((K, r), jnp.int32)
    for t in range(K):
        m = jnp.min(cv, axis=0, keepdims=True)
        cand = jnp.where(cv == m, ci, npts)
        j = jnp.min(cand, axis=0, keepdims=True)
        acc = jnp.where(kio == t, jnp.broadcast_to(j, (K, r)), acc)
        cv = jnp.where(cand == j, big, cv)
    out_ref[...] = acc


def _knn(coord):
    n = coord.shape[0]
    r = 128
    npad = 10240 if n == N_HIGH else ((n + 1279) // 1280) * 1280
    nqp = ((n + r - 1) // r) * r
    pad = jnp.full((npad - n, 3), 1e4, jnp.float32)
    cp0 = jnp.concatenate([coord, pad], axis=0)
    cp = jnp.concatenate([cp0, jnp.zeros((npad, 5), jnp.float32)], axis=1)
    qt = jnp.concatenate([cp0[:nqp].T, jnp.zeros((5, nqp), jnp.float32)], axis=0)
    sq = jnp.sum(cp0 * cp0, axis=1)[:, None]
    grid = nqp // r
    body = functools.partial(_knn_body, npts=npad, r=r)
    idx_t = pl.pallas_call(
        body,
        grid=(grid,),
        in_specs=[
            pl.BlockSpec((npad, 1), lambda i: (0, 0)),
            pl.BlockSpec((npad, 8), lambda i: (0, 0)),
            pl.BlockSpec((8, r), lambda i: (0, i)),
        ],
        out_specs=pl.BlockSpec((K, r), lambda i: (0, i)),
        out_shape=jax.ShapeDtypeStruct((K, nqp), jnp.int32),
    )(sq, cp, qt)
    return idx_t.T[:n]


NLP = 2560
NHP = 10240
TN = 128
NPAIR = N_HIGH * K
GRID_P = NHP // TN


def _bnm(y, gam, bet, nvalid):
    nr = y.shape[0]
    valid = jax.lax.broadcasted_iota(jnp.int32, (nr, 1), 0) < nvalid
    ym = jnp.where(valid, y, 0.0)
    mu = jnp.sum(ym, axis=0, keepdims=True) / nvalid
    dev = jnp.where(valid, y - mu, 0.0)
    var = jnp.sum(dev * dev, axis=0, keepdims=True) / nvalid
    return (y - mu) / jnp.sqrt(var + EPS) * gam + bet


def _pre_body(featp, skipp, wupt, bup, gup, betup, wskt, bsk, gsk, betsk,
              f0_ref, sf_ref):
    y0 = jax.lax.dot(featp[...], wupt[...],
                     preferred_element_type=jnp.float32) + bup[...]
    f0_ref[...] = jax.nn.relu(_bnm(y0, gup[...], betup[...], N_LOW))
    y1 = jax.lax.dot(skipp[...], wskt[...],
                     preferred_element_type=jnp.float32) + bsk[...]
    sf_ref[...] = jax.nn.relu(_bnm(y1, gsk[...], betsk[...], N_HIGH))


def _d1_body(fcl, sf, xyz8, wfc1t, g1, b1, wqt, bq, gq, bqn, wkt, bk, gk, bkn,
             wvt, bv, we1t, f_ref, v_ref, q8_ref, t_ref):
    f = fcl[...] + sf[...]
    f_ref[...] = f
    f1 = jax.nn.relu(_bnm(jax.lax.dot(f, wfc1t[...],
                                      preferred_element_type=jnp.float32),
                          g1[...], b1[...], N_HIGH))
    q = jax.nn.relu(_bnm(jax.lax.dot(f1, wqt[...],
                                     preferred_element_type=jnp.float32) + bq[...],
                         gq[...], bqn[...], N_HIGH))
    k = jax.nn.relu(_bnm(jax.lax.dot(f1, wkt[...],
                                     preferred_element_type=jnp.float32) + bk[...],
                         gk[...], bkn[...], N_HIGH))
    v_ref[...] = jax.lax.dot(f1, wvt[...],
                             preferred_element_type=jnp.float32) + bv[...]
    q8_ref[...] = jax.lax.dot(q, we1t[...], preferred_element_type=jnp.float32)
    k8 = jax.lax.dot(k, we1t[...], preferred_element_type=jnp.float32)
    t_ref[...] = jnp.concatenate(
        [xyz8[...][:, 0:3], k8, jnp.zeros((NHP, C - 11), jnp.float32)], axis=1)


def _pstat_body(g3, xyz8, o_ref):
    i = pl.program_id(0)
    pos = g3[...][:, :, 0:3] - xyz8[...][:, None, 0:3]
    valid = (jax.lax.broadcasted_iota(jnp.int32, (TN, 1, 1), 0)
             + i * TN) < N_HIGH
    pos = jnp.where(valid, pos, 0.0)
    pos2 = pos.reshape(TN * K, 3)
    s1 = jnp.sum(pos2, axis=0, keepdims=True)
    s2 = jax.lax.dot_general(pos2, pos2, (((0,), (0,)), ((), ())),
                             preferred_element_type=jnp.float32)
    part = jnp.concatenate([
        jnp.pad(s1, ((0, 0), (0, 125))),
        jnp.pad(s2, ((0, 0), (0, 125))),
        jnp.zeros((4, 128), jnp.float32)], axis=0)

    @pl.when(i == 0)
    def _():
        o_ref[...] = part

    @pl.when(i > 0)
    def _():
        o_ref[...] = o_ref[...] + part


def _pairs_front(g3, xyz8, q8, a128, a8p, ca, s128, m128, c8):
    del a128, s128
    pos = g3[:, :, 0:3] - xyz8[:, None, 0:3]
    h = jax.nn.relu(jax.lax.dot(pos.reshape(TN * K, 3), a8p[0:3, :],
                                preferred_element_type=jnp.float32) + ca)
    k8g = g3[:, :, 3:11].reshape(TN * K, 8)
    q8r = jnp.broadcast_to(q8[:, None, :], (TN, K, 8)).reshape(TN * K, 8)
    z8 = k8g - q8r + jax.lax.dot(h, m128,
                                 preferred_element_type=jnp.float32) + c8
    return h, z8


def _zstat_body(g3, xyz8, q8, a128, a8p, ca, s128, m128, c8, o_ref):
    i = pl.program_id(0)
    _, z8 = _pairs_front(g3[...], xyz8[...], q8[...], a128[...], a8p[...],
                         ca[...], s128[...], m128[...], c8[...])
    valid = (jax.lax.broadcasted_iota(jnp.int32, (TN * K, 1), 0)
             // K + i * TN) < N_HIGH
    z8 = jnp.where(valid, z8, 0.0)
    s1 = jnp.sum(z8, axis=0, keepdims=True)
    s2 = jnp.sum(z8 * z8, axis=0, keepdims=True)
    part = jnp.concatenate([
        jnp.pad(s1, ((0, 0), (0, 120))),
        jnp.pad(s2, ((0, 0), (0, 120))),
        jnp.zeros((6, 128), jnp.float32)], axis=0)

    @pl.when(i == 0)
    def _():
        o_ref[...] = part

    @pl.when(i > 0)
    def _():
        o_ref[...] = o_ref[...] + part


def _agg_body(g3, gv, xyz8, q8, a128, a8p, ca, s128, m128, c8, sw, tw,
              we2t, bwe2, p2t, bp2, e8, f2_ref):
    h, z8 = _pairs_front(g3[...], xyz8[...], q8[...], a128[...], a8p[...],
                         ca[...], s128[...], m128[...], c8[...])
    lg = jax.lax.dot(jax.nn.relu(z8 * sw[...] + tw[...]), we2t[...],
                     preferred_element_type=jnp.float32) + bwe2[...]
    l3 = lg.reshape(TN, K, 8)
    mx = jnp.max(l3, axis=1, keepdims=True)
    ex = jnp.exp(l3 - mx)
    w3 = ex / jnp.sum(ex, axis=1, keepdims=True)
    wexp = jax.lax.dot(w3.reshape(TN * K, 8), e8[...],
                       preferred_element_type=jnp.float32)
    peh = jax.lax.dot(h, p2t[...], preferred_element_type=jnp.float32)
    gv2 = gv[...].reshape(TN * K, C)
    seg = jnp.sum((wexp * (gv2 + peh)).reshape(TN, K, C), axis=1)
    f2_ref[...] = seg + bp2[...]


def _d3_body(f, f2, wfc3t, g2, b2, g3n, b3n, out_ref):
    y2 = jax.nn.relu(_bnm(f2[...], g2[...], b2[...], N_HIGH))
    y3 = _bnm(jax.lax.dot(y2, wfc3t[...], preferred_element_type=jnp.float32),
              g3n[...], b3n[...], N_HIGH)
    out_ref[...] = jax.nn.relu(f[...] + y3)


def _padr(x, rows):
    return jnp.concatenate(
        [x, jnp.zeros((rows - x.shape[0],) + x.shape[1:], x.dtype)], axis=0)


def _full_specs(shapes):
    def mk(n):
        return lambda i: (0,) * n
    return [pl.BlockSpec(s, mk(len(s))) for s in shapes]


_SC_NW = 32


def _gather_rows(table, idx):
    b = idx.shape[0]
    d = table.shape[1]
    bpw = b // _SC_NW
    bc = bpw
    while bc * d * 4 > 400_000:
        bc //= 2
    nch = bpw // bc
    mesh = plsc.VectorSubcoreMesh(core_axis_name="c", subcore_axis_name="s")

    @functools.partial(
        pl.kernel, mesh=mesh,
        out_type=jax.ShapeDtypeStruct((b, d), jnp.float32),
        scratch_types=[
            pltpu.VMEM((bc,), jnp.int32),
            pltpu.VMEM((bc, d), jnp.float32),
            pltpu.SemaphoreType.DMA,
        ],
    )
    def gk(table_hbm, idx_hbm, out_hbm, idx_v, rows_v, sem):
        wid = jax.lax.axis_index("s") * 2 + jax.lax.axis_index("c")
        for j in range(nch):
            base = wid * bpw + j * bc
            pltpu.sync_copy(idx_hbm.at[pl.ds(base, bc)], idx_v)
            pltpu.async_copy(table_hbm.at[idx_v], rows_v, sem).wait()
            pltpu.sync_copy(rows_v, out_hbm.at[pl.ds(base, bc)])

    return gk(table, idx)


def kernel(coord, feat, offset, skip_coord, skip_feat, skip_offset, cluster, params):
    ref = _knn(skip_coord)

    xyz8 = jnp.concatenate(
        [jnp.concatenate([skip_coord, jnp.full((NHP - N_HIGH, 3), 1e4,
                                               jnp.float32)], axis=0),
         jnp.zeros((NHP, 5), jnp.float32)], axis=1)
    idxfp = jnp.concatenate(
        [ref.reshape(-1),
         jnp.arange(NHP * K - NPAIR, dtype=jnp.int32) % N_HIGH])

    featp = _padr(feat, NLP)
    skipp = _padr(skip_feat, NHP)
    p = params
    r1 = lambda a: a.reshape(1, -1)
    f0, sf = pl.pallas_call(
        _pre_body,
        out_shape=[jax.ShapeDtypeStruct((NLP, C), jnp.float32),
                   jax.ShapeDtypeStruct((NHP, C), jnp.float32)],
    )(featp, skipp,
      p["up_proj"]["W"].T, r1(p["up_proj"]["b"]),
      r1(p["up_proj_bn"]["g"]), r1(p["up_proj_bn"]["b"]),
      p["up_skip"]["W"].T, r1(p["up_skip"]["b"]),
      r1(p["up_skip_bn"]["g"]), r1(p["up_skip_bn"]["b"]))

    clusterp = jnp.concatenate(
        [cluster, jnp.arange(NHP - N_HIGH, dtype=jnp.int32) % N_LOW])
    fcl = _gather_rows(f0, clusterp)

    f = None
    pstats = None
    for bi, blk in enumerate(p["blocks"]):
        we1t = blk["we1"]["W"].T
        if bi == 0:
            fin_a, fin_b = fcl, sf
        else:
            fin_a, fin_b = f, jnp.zeros_like(f)
        f, v, q8, tab = pl.pallas_call(
            _d1_body,
            out_shape=[jax.ShapeDtypeStruct((NHP, C), jnp.float32),
                       jax.ShapeDtypeStruct((NHP, C), jnp.float32),
                       jax.ShapeDtypeStruct((NHP, 8), jnp.float32),
                       jax.ShapeDtypeStruct((NHP, C), jnp.float32)],
        )(fin_a, fin_b, xyz8,
          blk["fc1"]["W"].T, r1(blk["norm1"]["g"]), r1(blk["norm1"]["b"]),
          blk["q"]["W"].T, r1(blk["q"]["b"]),
          r1(blk["q_bn"]["g"]), r1(blk["q_bn"]["b"]),
          blk["k"]["W"].T, r1(blk["k"]["b"]),
          r1(blk["k_bn"]["g"]), r1(blk["k_bn"]["b"]),
          blk["v"]["W"].T, r1(blk["v"]["b"]), we1t)

        g16 = _gather_rows(tab, idxfp).reshape(NHP, K, C)
        gv = _gather_rows(v, idxfp).reshape(NHP, K, C)

        if pstats is None:
            pstats = pl.pallas_call(
                _pstat_body,
                grid=(GRID_P,),
                in_specs=[pl.BlockSpec((TN, K, C), lambda i: (i, 0, 0)),
                          pl.BlockSpec((TN, 8), lambda i: (i, 0))],
                out_specs=pl.BlockSpec((8, 128), lambda i: (0, 0)),
                out_shape=jax.ShapeDtypeStruct((8, 128), jnp.float32),
            )(g16, xyz8)
            s1 = pstats[0, 0:3]
            s2 = pstats[1:4, 0:3]
            mpos = s1 / NPAIR
            cov = s2 / NPAIR - jnp.outer(mpos, mpos)

        p1t = blk["p1"]["W"].T
        mu_p = mpos @ p1t + blk["p1"]["b"]
        var_p = jnp.einsum("ij,ic,jc->c", cov, p1t, p1t)
        sg = blk["p_bn"]["g"] / jnp.sqrt(var_p + EPS)
        a3 = p1t * sg[None, :]
        a8p = jnp.concatenate([a3, jnp.zeros((5, C), jnp.float32)], axis=0)
        a128 = jnp.concatenate([a3, jnp.zeros((C - 3, C), jnp.float32)], axis=0)
        s128 = jnp.concatenate(
            [jnp.zeros((3, 8), jnp.float32), jnp.eye(8, dtype=jnp.float32),
             jnp.zeros((C - 11, 8), jnp.float32)], axis=0)
        ca = ((blk["p1"]["b"] - mu_p) * sg + blk["p_bn"]["b"]).reshape(1, C)
        m128 = blk["p2"]["W"].T @ we1t
        c8 = (blk["p2"]["b"] @ we1t + blk["we1"]["b"]).reshape(1, 8)

        zstats = pl.pallas_call(
            _zstat_body,
            grid=(GRID_P,),
            in_specs=[pl.BlockSpec((TN, K, C), lambda i: (i, 0, 0)),
                      pl.BlockSpec((TN, 8), lambda i: (i, 0)),
                      pl.BlockSpec((TN, 8), lambda i: (i, 0))]
                     + _full_specs([(C, C), (8, C), (1, C), (C, 8), (C, 8),
                                    (1, 8)]),
            out_specs=pl.BlockSpec((8, 128), lambda i: (0, 0)),
            out_shape=jax.ShapeDtypeStruct((8, 128), jnp.float32),
        )(g16, xyz8, q8, a128, a8p, ca, s128, m128, c8)
        mu_z = zstats[0, 0:8] / NPAIR
        var_z = zstats[1, 0:8] / NPAIR - mu_z * mu_z
        sw = (blk["we_bn"]["g"] / jnp.sqrt(var_z + EPS)).reshape(1, 8)
        tw = (blk["we_bn"]["b"] - mu_z * sw[0]).reshape(1, 8)

        e8 = (jnp.arange(C)[None, :] // 16 ==
              jnp.arange(G)[:, None]).astype(jnp.float32)
        f2 = pl.pallas_call(
            _agg_body,
            grid=(GRID_P,),
            in_specs=[pl.BlockSpec((TN, K, C), lambda i: (i, 0, 0)),
                      pl.BlockSpec((TN, K, C), lambda i: (i, 0, 0)),
                      pl.BlockSpec((TN, 8), lambda i: (i, 0)),
                      pl.BlockSpec((TN, 8), lambda i: (i, 0))]
                     + _full_specs([(C, C), (8, C), (1, C), (C, 8), (C, 8),
                                    (1, 8), (1, 8), (1, 8), (8, 8), (1, 8),
                                    (C, C), (1, C), (8, C)]),
            out_specs=pl.BlockSpec((TN, C), lambda i: (i, 0)),
            out_shape=jax.ShapeDtypeStruct((NHP, C), jnp.float32),
        )(g16, gv, xyz8, q8, a128, a8p, ca, s128, m128, c8, sw, tw,
          blk["we2"]["W"].T, r1(blk["we2"]["b"]), blk["p2"]["W"].T,
          r1(blk["p2"]["b"]), e8)

        f = pl.pallas_call(
            _d3_body,
            out_shape=jax.ShapeDtypeStruct((NHP, C), jnp.float32),
        )(f, f2, blk["fc3"]["W"].T,
          r1(blk["norm2"]["g"]), r1(blk["norm2"]["b"]),
          r1(blk["norm3"]["g"]), r1(blk["norm3"]["b"]))

    return (skip_coord, f[:N_HIGH], skip_offset, ref)
